# baseline (speedup 1.0000x reference)
import functools

import jax
import jax.numpy as jnp
from jax import lax
from jax.experimental import pallas as pl
from jax.experimental.pallas import tpu as pltpu
from jax.experimental.pallas import tpu_sc as plsc

M = 16384
K = 64
V = 8192
BETA_C = 0.25

MB = 2048
NB = 2048
LANES = 128
QCH = NB // LANES
BIG = 2 ** 30


def _dist_argmin_body(zsq_ref, esq_ref, z_ref, e2_ref, idx_ref, loss_ref,
                      val_ref, blk_ref, h1v_ref, h1i_ref, acc_ref):
    i = pl.program_id(0)
    j = pl.program_id(1)
    nj = pl.num_programs(1)

    @pl.when(jnp.logical_or(j == 0, j == nj // 2))
    def _():
        val_ref[...] = jnp.full((MB, LANES), jnp.inf, jnp.float32)

    dot2 = lax.dot_general(z_ref[...], e2_ref[...],
                           (((1,), (1,)), ((), ())),
                           preferred_element_type=jnp.float32)
    dist = (zsq_ref[...] + esq_ref[...]) + dot2

    v = val_ref[...]
    kk = blk_ref[...]
    for q in range(QCH):
        dch = dist[:, q * LANES:(q + 1) * LANES]
        better = dch < v
        v = jnp.where(better, dch, v)
        kk = jnp.where(better, (j * QCH + q).astype(jnp.float32), kk)
    val_ref[...] = v
    blk_ref[...] = kk

    def _resolve():
        vv = val_ref[...]
        lmin = jnp.min(vv, axis=1, keepdims=True)
        lane = lax.broadcasted_iota(jnp.int32, (MB, LANES), 1).astype(
            jnp.float32)
        cand = jnp.where(vv == lmin, blk_ref[...] * LANES + lane,
                         jnp.float32(BIG))
        gidx = jnp.min(cand, axis=1, keepdims=True)
        return lmin, gidx.astype(jnp.int32)

    @pl.when(j == nj // 2 - 1)
    def _():
        lmin, gidx = _resolve()
        h1v_ref[...] = lmin.astype(jnp.bfloat16).astype(jnp.float32)
        h1i_ref[...] = gidx

    @pl.when(j == nj - 1)
    def _():
        lmin2, gidx2 = _resolve()
        b2 = lmin2 < h1v_ref[...]
        fmin = jnp.where(b2, lmin2, h1v_ref[...])
        idx_ref[...] = jnp.where(b2, gidx2, h1i_ref[...])
        part = jnp.sum(fmin)

        @pl.when(i == 0)
        def _():
            acc_ref[0] = part

        @pl.when(i > 0)
        def _():
            acc_ref[0] = acc_ref[0] + part

        @pl.when(i == pl.num_programs(0) - 1)
        def _():
            c = acc_ref[0] / (M * K)
            loss_ref[...] = jnp.full((1, 1), c + BETA_C * c, jnp.float32)


_dist_argmin = pl.pallas_call(
    _dist_argmin_body,
    grid=(M // MB, V // NB),
    in_specs=[
        pl.BlockSpec((MB, 1), lambda i, j: (i, 0)),
        pl.BlockSpec((1, NB), lambda i, j: (0, j)),
        pl.BlockSpec((MB, K), lambda i, j: (i, 0)),
        pl.BlockSpec((NB, K), lambda i, j: (j, 0)),
    ],
    out_specs=[
        pl.BlockSpec((MB, 1), lambda i, j: (i, 0)),
        pl.BlockSpec((1, 1), lambda i, j: (0, 0)),
    ],
    out_shape=[
        jax.ShapeDtypeStruct((M, 1), jnp.int32),
        jax.ShapeDtypeStruct((1, 1), jnp.float32),
    ],
    scratch_shapes=[
        pltpu.VMEM((MB, LANES), jnp.float32),
        pltpu.VMEM((MB, LANES), jnp.float32),
        pltpu.VMEM((MB, 1), jnp.float32),
        pltpu.VMEM((MB, 1), jnp.int32),
        pltpu.SMEM((1,), jnp.float32),
    ],
)

_SC_CORES = 2
_SC_SUBCORES = 16
_NW = _SC_CORES * _SC_SUBCORES
_BPW = M // _NW


@functools.cache
def _make_sc_gather():
    @functools.partial(
        pl.kernel,
        out_type=jax.ShapeDtypeStruct((M, K), jnp.float32),
        mesh=plsc.VectorSubcoreMesh(core_axis_name="c",
                                    subcore_axis_name="s",
                                    num_cores=_SC_CORES,
                                    num_subcores=_SC_SUBCORES),
        scratch_types=[
            pltpu.VMEM((_BPW,), jnp.int32),
            pltpu.VMEM((_BPW, K), jnp.float32),
            pltpu.SemaphoreType.DMA,
        ],
        compiler_params=pltpu.CompilerParams(use_tc_tiling_on_sc=False),
    )
    def _sc_gather(embed_hbm, idx_hbm, out_hbm, idx_v, rows_v, sem):
        wid = lax.axis_index("s") * _SC_CORES + lax.axis_index("c")
        base = wid * _BPW
        pltpu.sync_copy(idx_hbm.at[pl.ds(base, _BPW)], idx_v)
        pltpu.async_copy(embed_hbm.at[idx_v], rows_v, sem).wait()
        pltpu.sync_copy(rows_v, out_hbm.at[pl.ds(base, _BPW)])

    return _sc_gather


def kernel(z_e, embed):
    z_sq = jnp.sum(z_e ** 2, axis=1, keepdims=True)
    e_sq = jnp.sum(embed ** 2, axis=1)[None, :]
    em2 = -2.0 * embed
    idx2d, loss2d = _dist_argmin(z_sq, e_sq, z_e, em2)
    indices = idx2d.reshape(M)
    z_q_st = _make_sc_gather()(embed, indices)
    vq_loss = loss2d[0, 0]
    return (z_q_st, indices, vq_loss)


# speedup vs baseline: 1.0215x; 1.0215x over previous
import functools

import jax
import jax.numpy as jnp
from jax import lax
from jax.experimental import pallas as pl
from jax.experimental.pallas import tpu as pltpu
from jax.experimental.pallas import tpu_sc as plsc

M = 16384
K = 64
V = 8192
BETA_C = 0.25

MB = 4096
NB = 1024
LANES = 128
QCH = NB // LANES
BIG = 2 ** 30


def _dist_argmin_body(zsq_ref, esq_ref, z_ref, e2_ref, idx_ref, loss_ref,
                      val_ref, blk_ref, h1v_ref, h1i_ref, acc_ref):
    i = pl.program_id(0)
    j = pl.program_id(1)
    nj = pl.num_programs(1)

    @pl.when(jnp.logical_or(j == 0, j == nj // 2))
    def _():
        val_ref[...] = jnp.full((MB, LANES), jnp.inf, jnp.float32)

    dot2 = lax.dot_general(z_ref[...], e2_ref[...],
                           (((1,), (1,)), ((), ())),
                           preferred_element_type=jnp.float32)
    dist = (zsq_ref[...] + esq_ref[...]) + dot2

    v = val_ref[...]
    kk = blk_ref[...]
    for q in range(QCH):
        dch = dist[:, q * LANES:(q + 1) * LANES]
        better = dch < v
        v = jnp.where(better, dch, v)
        kk = jnp.where(better, (j * QCH + q).astype(jnp.float32), kk)
    val_ref[...] = v
    blk_ref[...] = kk

    def _resolve():
        vv = val_ref[...]
        lmin = jnp.min(vv, axis=1, keepdims=True)
        lane = lax.broadcasted_iota(jnp.int32, (MB, LANES), 1).astype(
            jnp.float32)
        cand = jnp.where(vv == lmin, blk_ref[...] * LANES + lane,
                         jnp.float32(BIG))
        gidx = jnp.min(cand, axis=1, keepdims=True)
        return lmin, gidx.astype(jnp.int32)

    @pl.when(j == nj // 2 - 1)
    def _():
        lmin, gidx = _resolve()
        h1v_ref[...] = lmin.astype(jnp.bfloat16).astype(jnp.float32)
        h1i_ref[...] = gidx

    @pl.when(j == nj - 1)
    def _():
        lmin2, gidx2 = _resolve()
        b2 = lmin2 < h1v_ref[...]
        fmin = jnp.where(b2, lmin2, h1v_ref[...])
        idx_ref[...] = jnp.where(b2, gidx2, h1i_ref[...])
        part = jnp.sum(fmin)

        @pl.when(i == 0)
        def _():
            acc_ref[0] = part

        @pl.when(i > 0)
        def _():
            acc_ref[0] = acc_ref[0] + part

        @pl.when(i == pl.num_programs(0) - 1)
        def _():
            c = acc_ref[0] / (M * K)
            loss_ref[...] = jnp.full((1, 1), c + BETA_C * c, jnp.float32)


_dist_argmin = pl.pallas_call(
    _dist_argmin_body,
    grid=(M // MB, V // NB),
    in_specs=[
        pl.BlockSpec((MB, 1), lambda i, j: (i, 0)),
        pl.BlockSpec((1, NB), lambda i, j: (0, j)),
        pl.BlockSpec((MB, K), lambda i, j: (i, 0)),
        pl.BlockSpec((NB, K), lambda i, j: (j, 0)),
    ],
    out_specs=[
        pl.BlockSpec((MB, 1), lambda i, j: (i, 0)),
        pl.BlockSpec((1, 1), lambda i, j: (0, 0)),
    ],
    out_shape=[
        jax.ShapeDtypeStruct((M, 1), jnp.int32),
        jax.ShapeDtypeStruct((1, 1), jnp.float32),
    ],
    scratch_shapes=[
        pltpu.VMEM((MB, LANES), jnp.float32),
        pltpu.VMEM((MB, LANES), jnp.float32),
        pltpu.VMEM((MB, 1), jnp.float32),
        pltpu.VMEM((MB, 1), jnp.int32),
        pltpu.SMEM((1,), jnp.float32),
    ],
)

_SC_CORES = 2
_SC_SUBCORES = 16
_NW = _SC_CORES * _SC_SUBCORES
_BPW = M // _NW


@functools.cache
def _make_sc_gather():
    @functools.partial(
        pl.kernel,
        out_type=jax.ShapeDtypeStruct((M, K), jnp.float32),
        mesh=plsc.VectorSubcoreMesh(core_axis_name="c",
                                    subcore_axis_name="s",
                                    num_cores=_SC_CORES,
                                    num_subcores=_SC_SUBCORES),
        scratch_types=[
            pltpu.VMEM((_BPW,), jnp.int32),
            pltpu.VMEM((_BPW, K), jnp.float32),
            pltpu.SemaphoreType.DMA,
        ],
        compiler_params=pltpu.CompilerParams(use_tc_tiling_on_sc=False),
    )
    def _sc_gather(embed_hbm, idx_hbm, out_hbm, idx_v, rows_v, sem):
        wid = lax.axis_index("s") * _SC_CORES + lax.axis_index("c")
        base = wid * _BPW
        pltpu.sync_copy(idx_hbm.at[pl.ds(base, _BPW)], idx_v)
        pltpu.async_copy(embed_hbm.at[idx_v], rows_v, sem).wait()
        pltpu.sync_copy(rows_v, out_hbm.at[pl.ds(base, _BPW)])

    return _sc_gather


def kernel(z_e, embed):
    z_sq = jnp.sum(z_e ** 2, axis=1, keepdims=True)
    e_sq = jnp.sum(embed ** 2, axis=1)[None, :]
    em2 = -2.0 * embed
    idx2d, loss2d = _dist_argmin(z_sq, e_sq, z_e, em2)
    indices = idx2d.reshape(M)
    z_q_st = _make_sc_gather()(embed, indices)
    vq_loss = loss2d[0, 0]
    return (z_q_st, indices, vq_loss)
